# Initial kernel scaffold; baseline (speedup 1.0000x reference)
import jax
import jax.numpy as jnp
from jax.experimental import pallas as pl


def kernel(x, edge_index, batch, W1, b1, W2, b2, fc1_W, fc1_b, fc2_W, fc2_b):
    raise NotImplementedError("write your pallas kernel here")


# speedup vs baseline: 4.7992x; 4.7992x over previous
import functools

import jax
import jax.numpy as jnp
from jax import lax
from jax.experimental import pallas as pl
from jax.experimental.pallas import tpu as pltpu
from jax.experimental.pallas import tpu_sc as plsc

NC = 2
NS = 16
L = 16
K = 128

G = 1024


def _mesh():
    return plsc.VectorSubcoreMesh(
        core_axis_name="c", subcore_axis_name="s",
        num_cores=NC, num_subcores=NS)


_SC_PARAMS = pltpu.CompilerParams(use_tc_tiling_on_sc=False)


def _degree_pass(dst, np_rows):
    ep = dst.shape[0]
    per_t = ep // NS
    blocks = per_t // K
    half = np_rows // NC
    rows_per_tile = half // NS
    zb = rows_per_tile // K

    @functools.partial(
        pl.kernel,
        out_type=jax.ShapeDtypeStruct((np_rows, L), jnp.float32),
        mesh=_mesh(),
        scratch_types=[
            pltpu.VMEM((K,), jnp.int32),
            pltpu.VMEM((K,), jnp.int32),
            pltpu.VMEM((K, L), jnp.float32),
            pltpu.VMEM((K, L), jnp.float32),
            pltpu.VMEM_SHARED((half + 8, L), jnp.float32),
        ],
        compiler_params=_SC_PARAMS,
    )
    def body(dst_hbm, out_hbm, dst_v, dst2_v, ones_v, zero_v, acc_sh):
        cid = lax.axis_index("c")
        sid = lax.axis_index("s")
        base_row = sid * rows_per_tile
        node_base = cid * half

        @pl.loop(0, K)
        def _(i):
            ones_v[i] = jnp.ones((L,), jnp.float32)
            zero_v[i] = jnp.zeros((L,), jnp.float32)

        @pl.loop(0, zb)
        def _(j):
            pltpu.sync_copy(zero_v, acc_sh.at[pl.ds(base_row + j * K, K)])

        plsc.subcore_barrier()

        @pl.loop(0, blocks)
        def _(b):
            off = sid * per_t + b * K
            pltpu.sync_copy(dst_hbm.at[pl.ds(off, K)], dst_v)
            for j in range(K // L):
                d = dst_v[pl.ds(j * L, L)] - node_base
                ok = (d >= 0) & (d < half)
                dst2_v[pl.ds(j * L, L)] = jnp.where(ok, d, half)
            pltpu.sync_copy(ones_v, acc_sh.at[dst2_v], add=True)

        plsc.subcore_barrier()

        pltpu.sync_copy(acc_sh.at[pl.ds(base_row, rows_per_tile)],
                        out_hbm.at[pl.ds(node_base + base_row, rows_per_tile)])

    return body(dst)


def _agg_pass(src, dst, table, np_rows, nchunks):
    ep = src.shape[0]
    per_t = ep // NS
    blocks = per_t // K
    half = np_rows // NC
    rows_per_tile = half // NS
    zb = rows_per_tile // K

    @functools.partial(
        pl.kernel,
        out_type=jax.ShapeDtypeStruct((nchunks, np_rows, L), jnp.float32),
        mesh=_mesh(),
        scratch_types=[
            pltpu.VMEM((K,), jnp.int32),
            pltpu.VMEM((K,), jnp.int32),
            pltpu.VMEM((K,), jnp.int32),
            pltpu.VMEM((K, L), jnp.float32),
            pltpu.VMEM((K, L), jnp.float32),
            pltpu.VMEM_SHARED((half + 8, L), jnp.float32),
        ],
        compiler_params=_SC_PARAMS,
    )
    def body(src_hbm, dst_hbm, tab_hbm, out_hbm,
             src_v, idx_v, dst2_v, rows_v, zero_v, acc_sh):
        cid = lax.axis_index("c")
        sid = lax.axis_index("s")
        base_row = sid * rows_per_tile
        node_base = cid * half

        @pl.loop(0, K)
        def _(i):
            zero_v[i] = jnp.zeros((L,), jnp.float32)

        for c in range(nchunks):
            @pl.loop(0, zb)
            def _(j):
                pltpu.sync_copy(zero_v, acc_sh.at[pl.ds(base_row + j * K, K)])

            plsc.subcore_barrier()

            @pl.loop(0, blocks)
            def _(b):
                off = sid * per_t + b * K
                pltpu.sync_copy(src_hbm.at[pl.ds(off, K)], src_v)
                pltpu.sync_copy(dst_hbm.at[pl.ds(off, K)], dst2_v)
                for j in range(K // L):
                    sl = pl.ds(j * L, L)
                    if nchunks > 1:
                        idx_v[sl] = src_v[sl] * nchunks + c
                    else:
                        idx_v[sl] = src_v[sl]
                    d = dst2_v[sl] - node_base
                    ok = (d >= 0) & (d < half)
                    dst2_v[sl] = jnp.where(ok, d, half)
                pltpu.sync_copy(tab_hbm.at[idx_v], rows_v)
                pltpu.sync_copy(rows_v, acc_sh.at[dst2_v], add=True)

            plsc.subcore_barrier()

            pltpu.sync_copy(
                acc_sh.at[pl.ds(base_row, rows_per_tile)],
                out_hbm.at[c, pl.ds(node_base + base_row, rows_per_tile)])

            plsc.subcore_barrier()

    return body(src, dst, table)


def kernel(x, edge_index, batch, W1, b1, W2, b2, fc1_W, fc1_b, fc2_W, fc2_b):
    n = x.shape[0]
    e = edge_index.shape[1]
    np_rows = ((n + 4095) // 4096) * 4096
    ep = ((e + NS * K - 1) // (NS * K)) * (NS * K)

    pad_idx = jnp.full((ep - e,), np_rows - 1, jnp.int32)
    src = jnp.concatenate([edge_index[0], pad_idx])
    dst = jnp.concatenate([edge_index[1], pad_idx])
    x16 = jnp.pad(x, ((0, np_rows - n), (0, L - x.shape[1])))

    indeg = _degree_pass(dst, np_rows)
    dinv = lax.rsqrt(1.0 + indeg)
    xs = dinv * x16

    u0 = _agg_pass(src, dst, xs, np_rows, 1)[0]
    agg0 = dinv * (u0 + xs)
    W1p = jnp.pad(W1, ((0, L - W1.shape[0]), (0, 0)))
    h1 = jax.nn.relu(agg0 @ W1p + b1)
    h1s = dinv[:, :1] * h1

    u1 = _agg_pass(src, dst, h1s.reshape(4 * np_rows, L), np_rows, 4)
    u1 = u1.transpose(1, 0, 2).reshape(np_rows, 64)
    agg1 = dinv[:, :1] * (u1 + h1s)
    h2 = jax.nn.relu(agg1 @ W2 + b2)

    sums = jax.ops.segment_sum(h2[:n], batch, num_segments=G)
    counts = jax.ops.segment_sum(jnp.ones((n,), jnp.float32), batch,
                                 num_segments=G)
    g = sums / jnp.maximum(counts, 1.0)[:, None]
    g = jax.nn.relu(g @ fc1_W + fc1_b)
    return g @ fc2_W + fc2_b


# speedup vs baseline: 4.8576x; 1.0122x over previous
import functools

import jax
import jax.numpy as jnp
from jax import lax
from jax.experimental import pallas as pl
from jax.experimental.pallas import tpu as pltpu
from jax.experimental.pallas import tpu_sc as plsc

NC = 2
NS = 16
L = 16
K = 128

G = 1024


def _mesh():
    return plsc.VectorSubcoreMesh(
        core_axis_name="c", subcore_axis_name="s",
        num_cores=NC, num_subcores=NS)


_SC_PARAMS = pltpu.CompilerParams(use_tc_tiling_on_sc=False)


def _degree_pass(dst, np_rows):
    ep = dst.shape[0]
    per_t = ep // NS
    blocks = per_t // K
    half = np_rows // NC
    rows_per_tile = half // NS
    zb = rows_per_tile // K

    @functools.partial(
        pl.kernel,
        out_type=jax.ShapeDtypeStruct((np_rows, L), jnp.float32),
        mesh=_mesh(),
        scratch_types=[
            pltpu.VMEM((K,), jnp.int32),
            pltpu.VMEM((K,), jnp.int32),
            pltpu.VMEM((K, L), jnp.float32),
            pltpu.VMEM((K, L), jnp.float32),
            pltpu.VMEM_SHARED((half + 8, L), jnp.float32),
        ],
        compiler_params=_SC_PARAMS,
    )
    def body(dst_hbm, out_hbm, dst_v, dst2_v, ones_v, zero_v, acc_sh):
        cid = lax.axis_index("c")
        sid = lax.axis_index("s")
        base_row = sid * rows_per_tile
        node_base = cid * half

        @pl.loop(0, K)
        def _(i):
            ones_v[i] = jnp.ones((L,), jnp.float32)
            zero_v[i] = jnp.zeros((L,), jnp.float32)

        @pl.loop(0, zb)
        def _(j):
            pltpu.sync_copy(zero_v, acc_sh.at[pl.ds(base_row + j * K, K)])

        plsc.subcore_barrier()

        @pl.loop(0, blocks)
        def _(b):
            off = sid * per_t + b * K
            pltpu.sync_copy(dst_hbm.at[pl.ds(off, K)], dst_v)
            for j in range(K // L):
                d = dst_v[pl.ds(j * L, L)] - node_base
                ok = (d >= 0) & (d < half)
                dst2_v[pl.ds(j * L, L)] = jnp.where(ok, d, half)
            pltpu.sync_copy(ones_v, acc_sh.at[dst2_v], add=True)

        plsc.subcore_barrier()

        pltpu.sync_copy(acc_sh.at[pl.ds(base_row, rows_per_tile)],
                        out_hbm.at[pl.ds(node_base + base_row, rows_per_tile)])

    return body(dst)


def _agg_pass(src, dst, table, np_rows, nchunks):
    ep = src.shape[0]
    per_t = ep // NS
    blocks = per_t // K
    half = np_rows // NC
    rows_per_tile = half // NS
    zb = rows_per_tile // K

    @functools.partial(
        pl.kernel,
        out_type=jax.ShapeDtypeStruct((nchunks, np_rows, L), jnp.float32),
        mesh=_mesh(),
        scratch_types=[
            pltpu.VMEM((K,), jnp.int32),
            pltpu.VMEM((K,), jnp.int32),
            pltpu.VMEM((K,), jnp.int32),
            pltpu.VMEM((K, L), jnp.float32),
            pltpu.VMEM((K, L), jnp.float32),
            pltpu.VMEM_SHARED((half + 8, L), jnp.float32),
        ],
        compiler_params=_SC_PARAMS,
    )
    def body(src_hbm, dst_hbm, tab_hbm, out_hbm,
             src_v, idx_v, dst2_v, rows_v, zero_v, acc_sh):
        cid = lax.axis_index("c")
        sid = lax.axis_index("s")
        base_row = sid * rows_per_tile
        node_base = cid * half

        @pl.loop(0, K)
        def _(i):
            zero_v[i] = jnp.zeros((L,), jnp.float32)

        for c in range(nchunks):
            @pl.loop(0, zb)
            def _(j):
                pltpu.sync_copy(zero_v, acc_sh.at[pl.ds(base_row + j * K, K)])

            plsc.subcore_barrier()

            @pl.loop(0, blocks)
            def _(b):
                off = sid * per_t + b * K
                pltpu.sync_copy(src_hbm.at[pl.ds(off, K)], src_v)
                pltpu.sync_copy(dst_hbm.at[pl.ds(off, K)], dst2_v)
                for j in range(K // L):
                    sl = pl.ds(j * L, L)
                    if nchunks > 1:
                        idx_v[sl] = src_v[sl] * nchunks + c
                    else:
                        idx_v[sl] = src_v[sl]
                    d = dst2_v[sl] - node_base
                    ok = (d >= 0) & (d < half)
                    dst2_v[sl] = jnp.where(ok, d, half)
                pltpu.sync_copy(tab_hbm.at[idx_v], rows_v)
                pltpu.sync_copy(rows_v, acc_sh.at[dst2_v], add=True)

            plsc.subcore_barrier()

            pltpu.sync_copy(
                acc_sh.at[pl.ds(base_row, rows_per_tile)],
                out_hbm.at[c, pl.ds(node_base + base_row, rows_per_tile)])

            plsc.subcore_barrier()

    return body(src, dst, table)


def _dinv_xs(indeg, x16, np_rows):
    r8 = np_rows * L // 128
    rb = r8 // 8

    def body(deg_ref, x_ref, dinv_ref, xs_ref):
        dinv = lax.rsqrt(1.0 + deg_ref[...])
        dinv_ref[...] = dinv
        xs_ref[...] = dinv * x_ref[...]

    dinv, xs = pl.pallas_call(
        body,
        grid=(8,),
        in_specs=[pl.BlockSpec((rb, 128), lambda i: (i, 0)),
                  pl.BlockSpec((rb, 128), lambda i: (i, 0))],
        out_specs=[pl.BlockSpec((rb, 128), lambda i: (i, 0)),
                   pl.BlockSpec((rb, 128), lambda i: (i, 0))],
        out_shape=[jax.ShapeDtypeStruct((r8, 128), jnp.float32),
                   jax.ShapeDtypeStruct((r8, 128), jnp.float32)],
    )(indeg.reshape(r8, 128), x16.reshape(r8, 128))
    return dinv.reshape(np_rows, L), xs.reshape(np_rows, L)


_R = 1024


def _layer1(u0, xs, dinv, W1p, b1, np_rows):
    nb = np_rows // _R

    def body(u0_ref, xs_ref, dinv_ref, w_ref, b_ref, out_ref):
        dinv = dinv_ref[...]
        agg0 = dinv * (u0_ref[...] + xs_ref[...])
        h1 = jnp.maximum(jnp.dot(agg0, w_ref[...],
                                 preferred_element_type=jnp.float32,
                                 precision=lax.Precision.HIGHEST)
                         + b_ref[...], 0.0)
        out_ref[...] = dinv[:, :1] * h1

    return pl.pallas_call(
        body,
        grid=(nb,),
        in_specs=[pl.BlockSpec((_R, L), lambda i: (i, 0)),
                  pl.BlockSpec((_R, L), lambda i: (i, 0)),
                  pl.BlockSpec((_R, L), lambda i: (i, 0)),
                  pl.BlockSpec((L, 64), lambda i: (0, 0)),
                  pl.BlockSpec((1, 64), lambda i: (0, 0))],
        out_specs=pl.BlockSpec((_R, 64), lambda i: (i, 0)),
        out_shape=jax.ShapeDtypeStruct((np_rows, 64), jnp.float32),
    )(u0, xs, dinv, W1p, b1.reshape(1, 64))


def _layer2_pool_mlp(u1, h1s, dinv, batch3, W2, b2,
                     fc1_W, fc1_b, fc2_W, fc2_b, np_rows):
    nb = np_rows // _R

    def body(u1_ref, h1s_ref, dinv_ref, bt_ref, w2_ref, b2_ref,
             f1w_ref, f1b_ref, f2w_ref, f2b_ref, out_ref, sums, counts):
        i = pl.program_id(0)

        @pl.when(i == 0)
        def _():
            sums[...] = jnp.zeros_like(sums)
            counts[...] = jnp.zeros_like(counts)

        dinv1 = dinv_ref[...][:, :1]
        u1 = jnp.concatenate([u1_ref[c] for c in range(4)], axis=1)
        agg1 = dinv1 * (u1 + h1s_ref[...])
        z = jnp.maximum(jnp.dot(agg1, w2_ref[...],
                                preferred_element_type=jnp.float32,
                                 precision=lax.Precision.HIGHEST)
                        + b2_ref[...], 0.0)
        bt = bt_ref[0]
        oh = (lax.broadcasted_iota(jnp.int32, (G, _R), 0) == bt
              ).astype(jnp.float32)
        sums[...] += jnp.dot(oh, z, preferred_element_type=jnp.float32,
                                 precision=lax.Precision.HIGHEST)
        counts[...] += jnp.sum(oh, axis=1, keepdims=True)

        @pl.when(i == pl.num_programs(0) - 1)
        def _():
            gm = sums[...] / jnp.maximum(counts[...], 1.0)
            gg = jnp.maximum(jnp.dot(gm, f1w_ref[...],
                                     preferred_element_type=jnp.float32,
                                 precision=lax.Precision.HIGHEST)
                             + f1b_ref[...], 0.0)
            out_ref[...] = jnp.dot(gg, f2w_ref[...],
                                   preferred_element_type=jnp.float32,
                                   precision=lax.Precision.HIGHEST) + f2b_ref[...]

    return pl.pallas_call(
        body,
        grid=(nb,),
        in_specs=[pl.BlockSpec((4, _R, L), lambda i: (0, i, 0)),
                  pl.BlockSpec((_R, 64), lambda i: (i, 0)),
                  pl.BlockSpec((_R, L), lambda i: (i, 0)),
                  pl.BlockSpec((1, 1, _R), lambda i: (i, 0, 0)),
                  pl.BlockSpec((64, 128), lambda i: (0, 0)),
                  pl.BlockSpec((1, 128), lambda i: (0, 0)),
                  pl.BlockSpec((128, 64), lambda i: (0, 0)),
                  pl.BlockSpec((1, 64), lambda i: (0, 0)),
                  pl.BlockSpec((64, 1), lambda i: (0, 0)),
                  pl.BlockSpec((1, 1), lambda i: (0, 0))],
        out_specs=pl.BlockSpec((G, 1), lambda i: (0, 0)),
        out_shape=jax.ShapeDtypeStruct((G, 1), jnp.float32),
        scratch_shapes=[pltpu.VMEM((G, 128), jnp.float32),
                        pltpu.VMEM((G, 1), jnp.float32)],
    )(u1, h1s, dinv, batch3, W2, b2.reshape(1, 128),
      fc1_W, fc1_b.reshape(1, 64), fc2_W, fc2_b.reshape(1, 1))


def kernel(x, edge_index, batch, W1, b1, W2, b2, fc1_W, fc1_b, fc2_W, fc2_b):
    n = x.shape[0]
    e = edge_index.shape[1]
    np_rows = ((n + 4095) // 4096) * 4096
    ep = ((e + NS * K - 1) // (NS * K)) * (NS * K)

    pad_idx = jnp.full((ep - e,), np_rows - 1, jnp.int32)
    src = jnp.concatenate([edge_index[0], pad_idx])
    dst = jnp.concatenate([edge_index[1], pad_idx])
    x16 = jnp.pad(x, ((0, np_rows - n), (0, L - x.shape[1])))
    batch3 = jnp.pad(batch, (0, np_rows - n), constant_values=G
                     ).reshape(np_rows // _R, 1, _R)
    W1p = jnp.pad(W1, ((0, L - W1.shape[0]), (0, 0)))

    indeg = _degree_pass(dst, np_rows)
    dinv, xs = _dinv_xs(indeg, x16, np_rows)

    u0 = _agg_pass(src, dst, xs, np_rows, 1)[0]
    h1s = _layer1(u0, xs, dinv, W1p, b1, np_rows)

    u1 = _agg_pass(src, dst, h1s.reshape(4 * np_rows, L), np_rows, 4)
    return _layer2_pool_mlp(u1, h1s, dinv, batch3, W2, b2,
                            fc1_W, fc1_b, fc2_W, fc2_b, np_rows)
